# baseline (speedup 1.0000x reference)
import functools

import jax
import jax.numpy as jnp
from jax import lax
from jax.experimental import pallas as pl
from jax.experimental.pallas import tpu as pltpu
from jax.experimental.pallas import tpu_sc as plsc

N = 10000
E = 320000
D = 128
H = 64
K1 = 100
K2 = 25

NP = 10240
NTILES = 32
CH = 128
NCH = 80
EPT = CH * NCH
EP = NTILES * EPT
DUMMY = 10200
ROWS_PER_SUB = NP // 16

CH2 = 2048
NCH2 = 157
EP2 = CH2 * NCH2

_mesh = plsc.VectorSubcoreMesh(core_axis_name="c", subcore_axis_name="s")


@functools.partial(
    pl.kernel,
    mesh=_mesh,
    out_type=jax.ShapeDtypeStruct((NTILES, ROWS_PER_SUB, 16), jnp.float32),
    compiler_params=pltpu.CompilerParams(use_tc_tiling_on_sc=False),
    scratch_types=[
        pltpu.VMEM((NCH, CH), jnp.int32),
        pltpu.VMEM((CH, 16), jnp.float32),
        pltpu.VMEM_SHARED((NP, 16), jnp.float32),
        pltpu.SemaphoreType.DMA,
    ],
)
def _sc_deg(dst3, ones_hbm, zeros_hbm, out, idx_v, ones_v, acc, sem):
    c = lax.axis_index("c")
    s = lax.axis_index("s")
    wid = c * 16 + s
    pltpu.sync_copy(dst3.at[wid], idx_v)
    pltpu.sync_copy(ones_hbm, ones_v)
    pltpu.sync_copy(zeros_hbm, acc.at[pl.ds(s * ROWS_PER_SUB, ROWS_PER_SUB)])
    plsc.subcore_barrier()

    def body(j, carry):
        pltpu.sync_copy(ones_v, acc.at[idx_v.at[j]], add=True)
        return carry

    lax.fori_loop(0, NCH, body, 0)
    plsc.subcore_barrier()
    pltpu.sync_copy(acc.at[pl.ds(s * ROWS_PER_SUB, ROWS_PER_SUB)], out.at[wid])


@functools.partial(
    pl.kernel,
    mesh=_mesh,
    out_type=jax.ShapeDtypeStruct((NTILES, ROWS_PER_SUB, H), jnp.float32),
    compiler_params=pltpu.CompilerParams(use_tc_tiling_on_sc=False),
    scratch_types=[
        pltpu.VMEM((NCH, CH), jnp.int32),
        pltpu.VMEM((NCH, CH), jnp.int32),
        pltpu.VMEM((CH, H), jnp.float32),
        pltpu.VMEM((CH, H), jnp.float32),
        pltpu.VMEM_SHARED((NP, H), jnp.float32),
        pltpu.SemaphoreType.DMA,
        pltpu.SemaphoreType.DMA,
    ],
)
def _sc_agg(y_hbm, src3, dst3, zeros_hbm, out,
            sidx, didx, rows0, rows1, acc, sem0, sem1):
    c = lax.axis_index("c")
    s = lax.axis_index("s")
    wid = c * 16 + s
    pltpu.sync_copy(src3.at[wid], sidx)
    pltpu.sync_copy(dst3.at[wid], didx)
    pltpu.sync_copy(zeros_hbm, acc.at[pl.ds(s * ROWS_PER_SUB, ROWS_PER_SUB)])
    plsc.subcore_barrier()

    pltpu.async_copy(y_hbm.at[sidx.at[0]], rows0, sem0)

    def body(i, carry):
        j = 2 * i
        pltpu.async_copy(y_hbm.at[sidx.at[j + 1]], rows1, sem1)
        pltpu.make_async_copy(y_hbm.at[sidx.at[j]], rows0, sem0).wait()
        pltpu.sync_copy(rows0, acc.at[didx.at[j]], add=True)

        @pl.when(j + 2 < NCH)
        def _():
            pltpu.async_copy(y_hbm.at[sidx.at[j + 2]], rows0, sem0)

        pltpu.make_async_copy(y_hbm.at[sidx.at[j + 1]], rows1, sem1).wait()
        pltpu.sync_copy(rows1, acc.at[didx.at[j + 1]], add=True)
        return carry

    lax.fori_loop(0, NCH // 2, body, 0)
    plsc.subcore_barrier()
    pltpu.sync_copy(acc.at[pl.ds(s * ROWS_PER_SUB, ROWS_PER_SUB)], out.at[wid])


def _tc_pre_body(x_ref, w1_ref, degp_ref, y_ref, dinv_ref):
    deg = 1.0 + degp_ref[0, :, 0:1] + degp_ref[1, :, 0:1]
    dinv = lax.rsqrt(deg)
    xw = jnp.dot(x_ref[...], w1_ref[...], preferred_element_type=jnp.float32)
    y_ref[...] = xw * dinv
    dinv_ref[...] = dinv


def _tc_pre(x_pad, W1, degp):
    return pl.pallas_call(
        _tc_pre_body,
        out_shape=(
            jax.ShapeDtypeStruct((NP, H), jnp.float32),
            jax.ShapeDtypeStruct((NP, 1), jnp.float32),
        ),
    )(x_pad, W1, degp)


def _topk_loop(score, ii, k, vals_ref, perm_ref):
    def body(i, sc):
        gmax = jnp.max(sc)
        msk = sc == gmax
        idx = jnp.min(jnp.where(msk, ii, jnp.int32(2 ** 30)))
        vals_ref[pl.ds(i, 1), :] = gmax.reshape(1, 1)
        perm_ref[pl.ds(i, 1), :] = idx.reshape(1, 1)
        return jnp.where(ii == idx, jnp.float32(-2.0), sc)

    lax.fori_loop(0, k, body, score)
    return vals_ref[...], perm_ref[...]


def _tc_main_body(aggp_ref, y_ref, dinv_ref, src2_ref, dst2_ref,
                  b1_ref, p1_ref, w2_ref, b2_ref, p2_ref, w3_ref, b3_ref,
                  wl_ref, bl_ref, out_ref,
                  vals1_ref, perm1_ref, vals2_ref, perm2_ref):
    f32 = jnp.float32
    agg = aggp_ref[0] + aggp_ref[1]
    dinv = dinv_ref[...]
    h1 = jax.nn.relu(dinv * (agg + y_ref[...]) + b1_ref[...])

    p1 = p1_ref[...]
    rn1 = lax.rsqrt(jnp.sum(p1 * p1))
    score = jnp.tanh(jnp.dot(h1, p1.T, preferred_element_type=f32) * rn1)
    ii = lax.broadcasted_iota(jnp.int32, (NP, 1), 0)
    score = jnp.where(ii < N, score, f32(-2.0))

    vals1, perm1 = _topk_loop(score, ii, K1, vals1_ref, perm1_ref)

    iirow = lax.broadcasted_iota(jnp.int32, (1, NP), 1)
    oh1 = (perm1 == iirow).astype(f32)
    h1p = jnp.dot(oh1, h1, preferred_element_type=f32) * vals1
    xs1 = jnp.sum(h1p, axis=0, keepdims=True)

    def c2_body(j, acc):
        srow = src2_ref[pl.ds(j, 1), :]
        drow = dst2_ref[pl.ds(j, 1), :]
        ohs = (perm1 == srow).astype(f32)
        ohd = (perm1 == drow).astype(f32)
        return acc + lax.dot_general(
            ohd, ohs, (((1,), (1,)), ((), ())),
            preferred_element_type=f32)

    C2 = lax.fori_loop(0, NCH2, c2_body, jnp.zeros((K1, K1), f32))

    deg2 = 1.0 + jnp.sum(C2, axis=1, keepdims=True)
    dinv2 = lax.rsqrt(deg2)
    ia = lax.broadcasted_iota(jnp.int32, (K1, K1), 0)
    ib = lax.broadcasted_iota(jnp.int32, (K1, K1), 1)
    eye1 = (ia == ib).astype(f32)
    A2 = C2 * dinv2 * dinv2.T + eye1 * (dinv2 * dinv2)
    xw2 = jnp.dot(h1p, w2_ref[...], preferred_element_type=f32)
    h2 = jax.nn.relu(jnp.dot(A2, xw2, preferred_element_type=f32) + b2_ref[...])

    p2 = p2_ref[...]
    rn2 = lax.rsqrt(jnp.sum(p2 * p2))
    score2 = jnp.tanh(jnp.dot(h2, p2.T, preferred_element_type=f32) * rn2)
    ii2 = lax.broadcasted_iota(jnp.int32, (K1, 1), 0)
    vals2, perm2 = _topk_loop(score2, ii2, K2, vals2_ref, perm2_ref)

    ii2row = lax.broadcasted_iota(jnp.int32, (1, K1), 1)
    oh2 = (perm2 == ii2row).astype(f32)
    h2p = jnp.dot(oh2, h2, preferred_element_type=f32) * vals2
    xs2 = jnp.sum(h2p, axis=0, keepdims=True)

    C3 = jnp.dot(jnp.dot(oh2, C2, preferred_element_type=f32),
                 oh2.T, preferred_element_type=f32)
    deg3 = 1.0 + jnp.sum(C3, axis=1, keepdims=True)
    dinv3 = lax.rsqrt(deg3)
    ja = lax.broadcasted_iota(jnp.int32, (K2, K2), 0)
    jb = lax.broadcasted_iota(jnp.int32, (K2, K2), 1)
    eye2 = (ja == jb).astype(f32)
    A3 = C3 * dinv3 * dinv3.T + eye2 * (dinv3 * dinv3)
    xw3 = jnp.dot(h2p, w3_ref[...], preferred_element_type=f32)
    h3 = jax.nn.relu(jnp.dot(A3, xw3, preferred_element_type=f32) + b3_ref[...])
    xs3 = jnp.sum(h3, axis=0, keepdims=True)

    feat = jnp.concatenate([xs3, xs2, xs1], axis=1)
    out_ref[...] = jnp.dot(feat, wl_ref[...], preferred_element_type=f32) + bl_ref[...]


def _tc_main(aggp, y, dinv, src2, dst2, b1, p1, W2, b2, p2, W3, b3, Wl, bl):
    return pl.pallas_call(
        _tc_main_body,
        out_shape=jax.ShapeDtypeStruct((1, 2), jnp.float32),
        scratch_shapes=[
            pltpu.VMEM((K1, 1), jnp.float32),
            pltpu.VMEM((K1, 1), jnp.int32),
            pltpu.VMEM((K2, 1), jnp.float32),
            pltpu.VMEM((K2, 1), jnp.int32),
        ],
    )(aggp, y, dinv, src2, dst2, b1, p1, W2, b2, p2, W3, b3, Wl, bl)


def kernel(x, edge_index, batch, W1, b1, W2, b2, W3, b3, p1, p2, Wl, bl):
    src = edge_index[0].astype(jnp.int32)
    dst = edge_index[1].astype(jnp.int32)

    pad_sc = jnp.full((EP - E,), DUMMY, jnp.int32)
    src3 = jnp.concatenate([src, pad_sc]).reshape(NTILES, NCH, CH)
    dst3 = jnp.concatenate([dst, pad_sc]).reshape(NTILES, NCH, CH)
    pad2 = jnp.full((EP2 - E,), DUMMY, jnp.int32)
    src2 = jnp.concatenate([src, pad2]).reshape(NCH2, CH2)
    dst2 = jnp.concatenate([dst, pad2]).reshape(NCH2, CH2)

    x_pad = jnp.zeros((NP, D), jnp.float32).at[:N].set(x)

    ones16 = jnp.ones((CH, 16), jnp.float32)
    zeros16 = jnp.zeros((ROWS_PER_SUB, 16), jnp.float32)
    zerosH = jnp.zeros((ROWS_PER_SUB, H), jnp.float32)

    degp = _sc_deg(dst3, ones16, zeros16)
    degp = degp.reshape(2, NP, 16)
    y, dinv = _tc_pre(x_pad, W1, degp)
    aggp = _sc_agg(y, src3, dst3, zerosH)
    aggp = aggp.reshape(2, NP, H)

    return _tc_main(aggp, y, dinv, src2, dst2,
                    b1.reshape(1, H), p1.reshape(1, H),
                    W2, b2.reshape(1, H), p2.reshape(1, H),
                    W3, b3.reshape(1, H),
                    Wl, bl.reshape(1, 2))


# speedup vs baseline: 1.3137x; 1.3137x over previous
import functools

import jax
import jax.numpy as jnp
from jax import lax
from jax.experimental import pallas as pl
from jax.experimental.pallas import tpu as pltpu
from jax.experimental.pallas import tpu_sc as plsc

N = 10000
E = 320000
D = 128
H = 64
K1 = 100
K2 = 25

NP = 10240
NTILES = 32
CH = 128
NCH = 80
EPT = CH * NCH
EP = NTILES * EPT
DUMMY = 10200
ROWS_PER_SUB = NP // 16

CH2 = 2048
NCH2 = 157
EP2 = CH2 * NCH2

_mesh = plsc.VectorSubcoreMesh(core_axis_name="c", subcore_axis_name="s")


@functools.partial(
    pl.kernel,
    mesh=_mesh,
    out_type=jax.ShapeDtypeStruct((NTILES, ROWS_PER_SUB, 16), jnp.float32),
    compiler_params=pltpu.CompilerParams(use_tc_tiling_on_sc=False),
    scratch_types=[
        pltpu.VMEM((NCH, CH), jnp.int32),
        pltpu.VMEM((CH, 16), jnp.float32),
        pltpu.VMEM_SHARED((NP, 16), jnp.float32),
        pltpu.SemaphoreType.DMA,
    ],
)
def _sc_deg(dst3, ones_hbm, zeros_hbm, out, idx_v, ones_v, acc, sem):
    c = lax.axis_index("c")
    s = lax.axis_index("s")
    wid = c * 16 + s
    pltpu.sync_copy(dst3.at[wid], idx_v)
    pltpu.sync_copy(ones_hbm, ones_v)
    pltpu.sync_copy(zeros_hbm, acc.at[pl.ds(s * ROWS_PER_SUB, ROWS_PER_SUB)])
    plsc.subcore_barrier()

    def body(j, carry):
        pltpu.sync_copy(ones_v, acc.at[idx_v.at[j]], add=True)
        return carry

    lax.fori_loop(0, NCH, body, 0)
    plsc.subcore_barrier()
    pltpu.sync_copy(acc.at[pl.ds(s * ROWS_PER_SUB, ROWS_PER_SUB)], out.at[wid])


@functools.partial(
    pl.kernel,
    mesh=_mesh,
    out_type=jax.ShapeDtypeStruct((NTILES, ROWS_PER_SUB, H), jnp.float32),
    compiler_params=pltpu.CompilerParams(use_tc_tiling_on_sc=False),
    scratch_types=[
        pltpu.VMEM((NCH, CH), jnp.int32),
        pltpu.VMEM((NCH, CH), jnp.int32),
        pltpu.VMEM((CH, H), jnp.float32),
        pltpu.VMEM((CH, H), jnp.float32),
        pltpu.VMEM_SHARED((NP, H), jnp.float32),
        pltpu.VMEM_SHARED((NP, H), jnp.float32),
        pltpu.SemaphoreType.DMA,
        pltpu.SemaphoreType.DMA,
    ],
)
def _sc_agg(y_hbm, src3, dst3, zeros_hbm, out,
            sidx, didx, rows0, rows1, acc, y_sh, sem0, sem1):
    c = lax.axis_index("c")
    s = lax.axis_index("s")
    wid = c * 16 + s
    row_slc = pl.ds(s * ROWS_PER_SUB, ROWS_PER_SUB)
    pltpu.sync_copy(src3.at[wid], sidx)
    pltpu.sync_copy(dst3.at[wid], didx)
    pltpu.sync_copy(y_hbm.at[row_slc], y_sh.at[row_slc])
    pltpu.sync_copy(zeros_hbm, acc.at[row_slc])
    plsc.subcore_barrier()

    pltpu.async_copy(y_sh.at[sidx.at[0]], rows0, sem0)

    def body(i, carry):
        j = 2 * i
        pltpu.async_copy(y_sh.at[sidx.at[j + 1]], rows1, sem1)
        pltpu.make_async_copy(y_sh.at[sidx.at[j]], rows0, sem0).wait()
        pltpu.sync_copy(rows0, acc.at[didx.at[j]], add=True)

        @pl.when(j + 2 < NCH)
        def _():
            pltpu.async_copy(y_sh.at[sidx.at[j + 2]], rows0, sem0)

        pltpu.make_async_copy(y_sh.at[sidx.at[j + 1]], rows1, sem1).wait()
        pltpu.sync_copy(rows1, acc.at[didx.at[j + 1]], add=True)
        return carry

    lax.fori_loop(0, NCH // 2, body, 0)
    plsc.subcore_barrier()
    pltpu.sync_copy(acc.at[pl.ds(s * ROWS_PER_SUB, ROWS_PER_SUB)], out.at[wid])


def _tc_pre_body(x_ref, w1_ref, degp_ref, y_ref, dinv_ref):
    deg = 1.0 + degp_ref[0, :, 0:1] + degp_ref[1, :, 0:1]
    dinv = lax.rsqrt(deg)
    xw = jnp.dot(x_ref[...], w1_ref[...], preferred_element_type=jnp.float32)
    y_ref[...] = xw * dinv
    dinv_ref[...] = dinv


def _tc_pre(x_pad, W1, degp):
    return pl.pallas_call(
        _tc_pre_body,
        out_shape=(
            jax.ShapeDtypeStruct((NP, H), jnp.float32),
            jax.ShapeDtypeStruct((NP, 1), jnp.float32),
        ),
    )(x_pad, W1, degp)


def _topk_loop(score, ii, k, vals_ref, perm_ref):
    def body(i, sc):
        gmax = jnp.max(sc)
        msk = sc == gmax
        idx = jnp.min(jnp.where(msk, ii, jnp.int32(2 ** 30)))
        vals_ref[pl.ds(i, 1), :] = gmax.reshape(1, 1)
        perm_ref[pl.ds(i, 1), :] = idx.reshape(1, 1)
        return jnp.where(ii == idx, jnp.float32(-2.0), sc)

    lax.fori_loop(0, k, body, score)
    return vals_ref[...], perm_ref[...]


def _tc_main_body(aggp_ref, y_ref, dinv_ref, src2_ref, dst2_ref,
                  b1_ref, p1_ref, w2_ref, b2_ref, p2_ref, w3_ref, b3_ref,
                  wl_ref, bl_ref, out_ref,
                  vals1_ref, perm1_ref, vals2_ref, perm2_ref):
    f32 = jnp.float32
    agg = aggp_ref[0] + aggp_ref[1]
    dinv = dinv_ref[...]
    h1 = jax.nn.relu(dinv * (agg + y_ref[...]) + b1_ref[...])

    p1 = p1_ref[...]
    rn1 = lax.rsqrt(jnp.sum(p1 * p1))
    score = jnp.tanh(jnp.dot(h1, p1.T, preferred_element_type=f32) * rn1)
    ii = lax.broadcasted_iota(jnp.int32, (NP, 1), 0)
    score = jnp.where(ii < N, score, f32(-2.0))

    vals1, perm1 = _topk_loop(score, ii, K1, vals1_ref, perm1_ref)

    iirow = lax.broadcasted_iota(jnp.int32, (1, NP), 1)
    oh1 = (perm1 == iirow).astype(f32)
    h1p = jnp.dot(oh1, h1, preferred_element_type=f32) * vals1
    xs1 = jnp.sum(h1p, axis=0, keepdims=True)

    def c2_body(j, acc):
        srow = src2_ref[pl.ds(j, 1), :]
        drow = dst2_ref[pl.ds(j, 1), :]
        ohs = (perm1 == srow).astype(f32)
        ohd = (perm1 == drow).astype(f32)
        return acc + lax.dot_general(
            ohd, ohs, (((1,), (1,)), ((), ())),
            preferred_element_type=f32)

    C2 = lax.fori_loop(0, NCH2, c2_body, jnp.zeros((K1, K1), f32))

    deg2 = 1.0 + jnp.sum(C2, axis=1, keepdims=True)
    dinv2 = lax.rsqrt(deg2)
    ia = lax.broadcasted_iota(jnp.int32, (K1, K1), 0)
    ib = lax.broadcasted_iota(jnp.int32, (K1, K1), 1)
    eye1 = (ia == ib).astype(f32)
    A2 = C2 * dinv2 * dinv2.T + eye1 * (dinv2 * dinv2)
    xw2 = jnp.dot(h1p, w2_ref[...], preferred_element_type=f32)
    h2 = jax.nn.relu(jnp.dot(A2, xw2, preferred_element_type=f32) + b2_ref[...])

    p2 = p2_ref[...]
    rn2 = lax.rsqrt(jnp.sum(p2 * p2))
    score2 = jnp.tanh(jnp.dot(h2, p2.T, preferred_element_type=f32) * rn2)
    ii2 = lax.broadcasted_iota(jnp.int32, (K1, 1), 0)
    vals2, perm2 = _topk_loop(score2, ii2, K2, vals2_ref, perm2_ref)

    ii2row = lax.broadcasted_iota(jnp.int32, (1, K1), 1)
    oh2 = (perm2 == ii2row).astype(f32)
    h2p = jnp.dot(oh2, h2, preferred_element_type=f32) * vals2
    xs2 = jnp.sum(h2p, axis=0, keepdims=True)

    C3 = jnp.dot(jnp.dot(oh2, C2, preferred_element_type=f32),
                 oh2.T, preferred_element_type=f32)
    deg3 = 1.0 + jnp.sum(C3, axis=1, keepdims=True)
    dinv3 = lax.rsqrt(deg3)
    ja = lax.broadcasted_iota(jnp.int32, (K2, K2), 0)
    jb = lax.broadcasted_iota(jnp.int32, (K2, K2), 1)
    eye2 = (ja == jb).astype(f32)
    A3 = C3 * dinv3 * dinv3.T + eye2 * (dinv3 * dinv3)
    xw3 = jnp.dot(h2p, w3_ref[...], preferred_element_type=f32)
    h3 = jax.nn.relu(jnp.dot(A3, xw3, preferred_element_type=f32) + b3_ref[...])
    xs3 = jnp.sum(h3, axis=0, keepdims=True)

    feat = jnp.concatenate([xs3, xs2, xs1], axis=1)
    out_ref[...] = jnp.dot(feat, wl_ref[...], preferred_element_type=f32) + bl_ref[...]


def _tc_main(aggp, y, dinv, src2, dst2, b1, p1, W2, b2, p2, W3, b3, Wl, bl):
    return pl.pallas_call(
        _tc_main_body,
        out_shape=jax.ShapeDtypeStruct((1, 2), jnp.float32),
        scratch_shapes=[
            pltpu.VMEM((K1, 1), jnp.float32),
            pltpu.VMEM((K1, 1), jnp.int32),
            pltpu.VMEM((K2, 1), jnp.float32),
            pltpu.VMEM((K2, 1), jnp.int32),
        ],
    )(aggp, y, dinv, src2, dst2, b1, p1, W2, b2, p2, W3, b3, Wl, bl)


def kernel(x, edge_index, batch, W1, b1, W2, b2, W3, b3, p1, p2, Wl, bl):
    src = edge_index[0].astype(jnp.int32)
    dst = edge_index[1].astype(jnp.int32)

    pad_sc = jnp.full((EP - E,), DUMMY, jnp.int32)
    src3 = jnp.concatenate([src, pad_sc]).reshape(NTILES, NCH, CH)
    dst3 = jnp.concatenate([dst, pad_sc]).reshape(NTILES, NCH, CH)
    pad2 = jnp.full((EP2 - E,), DUMMY, jnp.int32)
    src2 = jnp.concatenate([src, pad2]).reshape(NCH2, CH2)
    dst2 = jnp.concatenate([dst, pad2]).reshape(NCH2, CH2)

    x_pad = jnp.zeros((NP, D), jnp.float32).at[:N].set(x)

    ones16 = jnp.ones((CH, 16), jnp.float32)
    zeros16 = jnp.zeros((ROWS_PER_SUB, 16), jnp.float32)
    zerosH = jnp.zeros((ROWS_PER_SUB, H), jnp.float32)

    degp = _sc_deg(dst3, ones16, zeros16)
    degp = degp.reshape(2, NP, 16)
    y, dinv = _tc_pre(x_pad, W1, degp)
    aggp = _sc_agg(y, src3, dst3, zerosH)
    aggp = aggp.reshape(2, NP, H)

    return _tc_main(aggp, y, dinv, src2, dst2,
                    b1.reshape(1, H), p1.reshape(1, H),
                    W2, b2.reshape(1, H), p2.reshape(1, H),
                    W3, b3.reshape(1, H),
                    Wl, bl.reshape(1, 2))
